# Initial kernel scaffold; baseline (speedup 1.0000x reference)
import functools

import jax
import jax.numpy as jnp
from jax import lax
from jax.experimental import pallas as pl
from jax.experimental.pallas import tpu as pltpu
from jax.experimental.pallas import tpu_sc as plsc

_NC = 2
_NS = 16
_L = 16
_K = 128


def _sc_spmm(x, src1, dst1, val1, src2, dst2, val2):
    n, d = x.shape
    epad = src1.shape[0]
    pt = epad // _NS
    chunks = pt // _K
    npt = n // _NS
    assert pt % _K == 0 and npt * _NS == n and d % _L == 0

    mesh = plsc.VectorSubcoreMesh(core_axis_name="c", subcore_axis_name="s")

    @functools.partial(
        pl.kernel,
        mesh=mesh,
        out_type=(
            jax.ShapeDtypeStruct((n, d), jnp.float32),
            jax.ShapeDtypeStruct((n, d), jnp.float32),
        ),
        scratch_types=[
            pltpu.VMEM((_K,), jnp.int32),
            pltpu.VMEM((_K,), jnp.int32),
            pltpu.VMEM((_K,), jnp.float32),
            pltpu.VMEM((_K, d), jnp.float32),
            pltpu.VMEM((n // _NS, d), jnp.float32),
            pltpu.SemaphoreType.DMA,
            pltpu.VMEM_SHARED((n, d), jnp.float32),
        ],
    )
    def spmm_kernel(x_hbm, s1, d1, v1, s2, d2, v2, g1, g2,
                    src_v, dst_v, val_v, rows_v, buf_v, sem, acc):
        cid = lax.axis_index("c")
        tid = lax.axis_index("s")

        zero = jnp.zeros((_L,), jnp.float32)

        @pl.loop(0, npt)
        def _(r):
            for j in range(d // _L):
                buf_v[r, pl.ds(j * _L, _L)] = zero

        def process(src_hbm, dst_hbm, valr_hbm, out_hbm):
            row0 = tid * npt
            pltpu.sync_copy(buf_v, acc.at[pl.ds(row0, npt)])
            plsc.subcore_barrier()

            @pl.loop(0, chunks)
            def _(c):
                base = pl.multiple_of(tid * pt + c * _K, _K)
                pltpu.sync_copy(src_hbm.at[pl.ds(base, _K)], src_v)
                pltpu.sync_copy(dst_hbm.at[pl.ds(base, _K)], dst_v)
                pltpu.sync_copy(valr_hbm.at[pl.ds(base, _K)], val_v)
                pltpu.async_copy(x_hbm.at[src_v], rows_v, sem).wait()

                @pl.loop(0, _K)
                def _(e):
                    vs = plsc.load_gather(val_v, [jnp.full((_L,), e, jnp.int32)])
                    for j in range(d // _L):
                        sl = pl.ds(j * _L, _L)
                        rows_v[e, sl] = rows_v[e, sl] * vs

                pltpu.sync_copy(rows_v, acc.at[dst_v], add=True)

            plsc.subcore_barrier()
            pltpu.sync_copy(acc.at[pl.ds(row0, npt)], buf_v)
            pltpu.sync_copy(buf_v, out_hbm.at[pl.ds(row0, npt)])

        @pl.when(cid == 0)
        def _():
            process(s1, d1, v1, g1)

        @pl.when(cid == 1)
        def _():
            process(s2, d2, v2, g2)

    return spmm_kernel(x, src1, dst1, val1, src2, dst2, val2)


def _ln_body(x_ref, g1_ref, g2_ref, w_ref, scale_ref, bias_ref, out_ref):
    w = w_ref[...]
    h0 = jnp.dot(x_ref[...], w, preferred_element_type=jnp.float32)
    h1 = jnp.dot(g1_ref[...], w, preferred_element_type=jnp.float32)
    h2 = jnp.dot(g2_ref[...], w, preferred_element_type=jnp.float32)
    h = jnp.concatenate([h0, h1, h2], axis=1)
    mu = jnp.mean(h, axis=1, keepdims=True)
    var = jnp.mean(h * h, axis=1, keepdims=True) - mu * mu
    inv = lax.rsqrt(var + 1e-5)
    out_ref[...] = (h - mu) * inv * scale_ref[...] + bias_ref[...]


def _ln_tc(x, g1, g2, w, ln_scale, ln_bias):
    n, d = x.shape
    out_dim = ln_scale.shape[0]
    bt = 1000
    grid = n // bt
    return pl.pallas_call(
        _ln_body,
        grid=(grid,),
        in_specs=[
            pl.BlockSpec((bt, d), lambda i: (i, 0)),
            pl.BlockSpec((bt, d), lambda i: (i, 0)),
            pl.BlockSpec((bt, d), lambda i: (i, 0)),
            pl.BlockSpec((d, d), lambda i: (0, 0)),
            pl.BlockSpec((1, out_dim), lambda i: (0, 0)),
            pl.BlockSpec((1, out_dim), lambda i: (0, 0)),
        ],
        out_specs=pl.BlockSpec((bt, out_dim), lambda i: (i, 0)),
        out_shape=jax.ShapeDtypeStruct((n, out_dim), jnp.float32),
    )(x, g1, g2, w, ln_scale.reshape(1, -1), ln_bias.reshape(1, -1))


def kernel(x, adj1_indices, adj1_values, adj2_indices, adj2_values,
           W, ln_scale, ln_bias):
    e = adj1_values.shape[0]
    per_tile = -(-e // _NS)
    per_tile = -(-per_tile // _K) * _K
    epad = per_tile * _NS
    pad = epad - e

    def prep(indices, values):
        dst = indices[0]
        src = indices[1]
        if pad:
            zi = jnp.zeros((pad,), jnp.int32)
            src = jnp.concatenate([src, zi])
            dst = jnp.concatenate([dst, zi])
            values = jnp.concatenate([values, jnp.zeros((pad,), jnp.float32)])
        return src, dst, values

    s1, d1, v1 = prep(adj1_indices, adj1_values)
    s2, d2, v2 = prep(adj2_indices, adj2_values)
    g1, g2 = _sc_spmm(x, s1, d1, v1, s2, d2, v2)
    return _ln_tc(x, g1, g2, W, ln_scale, ln_bias)


# speedup vs baseline: 3.6205x; 3.6205x over previous
import functools

import jax
import jax.numpy as jnp
from jax import lax
from jax.experimental import pallas as pl
from jax.experimental.pallas import tpu as pltpu
from jax.experimental.pallas import tpu_sc as plsc

_NC = 2
_NS = 16
_L = 16
_K = 128

_SPLAT_DNUMS = lax.GatherDimensionNumbers(
    offset_dims=(), collapsed_slice_dims=(0,), start_index_map=(0,))


_G = 8


def _sc_spmm(x, idx1, val1, idx2, val2):
    n, d = x.shape
    chunks_total = idx1.shape[0]
    pt = (chunks_total // _NS) * _K
    groups = chunks_total // (_NS * _G)
    npt = -(-n // (_NS * _K)) * _K
    n_pad = npt * _NS
    assert chunks_total == groups * _NS * _G and d % _L == 0

    mesh = plsc.VectorSubcoreMesh(core_axis_name="c", subcore_axis_name="s")

    @functools.partial(
        pl.kernel,
        mesh=mesh,
        out_type=(
            jax.ShapeDtypeStruct((n_pad, d), jnp.float32),
            jax.ShapeDtypeStruct((n_pad, d), jnp.float32),
        ),
        scratch_types=[
            pltpu.VMEM((_G, 2, _K), jnp.int32),
            pltpu.VMEM((_G, _K), jnp.float32),
            pltpu.VMEM((_K, d), jnp.float32),
            pltpu.VMEM((_K, d), jnp.float32),
            pltpu.SemaphoreType.DMA,
            pltpu.SemaphoreType.DMA,
            pltpu.SemaphoreType.DMA,
            pltpu.SemaphoreType.DMA,
            pltpu.VMEM_SHARED((n_pad, d), jnp.float32),
        ],
    )
    def spmm_kernel(x_hbm, i1, v1, i2, v2, g1, g2,
                    idx_v, val_v, r0, r1, gs0, gs1, ss0, ss1, acc):
        cid = lax.axis_index("c")
        tid = lax.axis_index("s")
        rows = (r0, r1)
        gsem = (gs0, gs1)
        ssem = (ss0, ss1)

        zero = jnp.zeros((_L,), jnp.float32)

        @pl.loop(0, _K)
        def _(r):
            for j in range(d // _L):
                r0[r, pl.ds(j * _L, _L)] = zero

        def gather_start(b, j):
            pltpu.async_copy(x_hbm.at[idx_v.at[j, 0]], rows[b], gsem[b])

        def gather_wait(b):
            pltpu.make_async_copy(
                x_hbm.at[idx_v.at[0, 0]], rows[b], gsem[b]).wait()

        def scatter_start(b, j):
            pltpu.async_copy(rows[b], acc.at[idx_v.at[j, 1]], ssem[b],
                             add=True)

        def scatter_wait(b):
            pltpu.make_async_copy(
                rows[b], acc.at[idx_v.at[0, 1]], ssem[b]).wait()

        def scale(b, j):
            rb = rows[b]

            @pl.loop(0, _K // _L)
            def _(gg):
                gbase = gg * _L
                val16 = val_v[j, pl.ds(gbase, _L)]
                for l in range(_L):
                    vs = lax.gather(
                        val16, jnp.full((_L, 1), l, jnp.int32),
                        _SPLAT_DNUMS, (1,),
                        mode=lax.GatherScatterMode.PROMISE_IN_BOUNDS)
                    for jj in range(d // _L):
                        sl = pl.ds(jj * _L, _L)
                        rb[gbase + l, sl] = rb[gbase + l, sl] * vs

        def process(i_hbm, v_hbm, out_hbm):
            row0 = tid * npt
            for i in range(npt // _K):
                pltpu.sync_copy(r0, acc.at[pl.ds(row0 + i * _K, _K)])
            plsc.subcore_barrier()

            gchunk0 = tid * (groups * _G)

            @pl.loop(0, groups)
            def _(g):
                @pl.when(g > 0)
                def _():
                    scatter_wait(0)
                    scatter_wait(1)

                row = pl.ds(gchunk0 + g * _G, _G)
                pltpu.sync_copy(i_hbm.at[row], idx_v)
                pltpu.sync_copy(v_hbm.at[row], val_v)
                gather_start(0, 0)
                gather_start(1, 1)

                @pl.loop(0, _G, step=2)
                def _(j):
                    gather_wait(0)
                    scale(0, j)
                    scatter_start(0, j)
                    gather_wait(1)
                    scale(1, j + 1)
                    scatter_start(1, j + 1)

                    @pl.when(j + 2 < _G)
                    def _():
                        scatter_wait(0)
                        gather_start(0, j + 2)
                        scatter_wait(1)
                        gather_start(1, j + 3)

            scatter_wait(0)
            scatter_wait(1)
            plsc.subcore_barrier()
            for i in range(npt // _K):
                pltpu.sync_copy(acc.at[pl.ds(row0 + i * _K, _K)], r0)
                pltpu.sync_copy(r0, out_hbm.at[pl.ds(row0 + i * _K, _K)])

        @pl.when(cid == 0)
        def _():
            process(i1, v1, g1)

        @pl.when(cid == 1)
        def _():
            process(i2, v2, g2)

    return spmm_kernel(x, idx1, val1, idx2, val2)


def _ln_body(x_ref, g1_ref, g2_ref, w_ref, scale_ref, bias_ref, out_ref):
    w = w_ref[...]
    h0 = jnp.dot(x_ref[...], w, preferred_element_type=jnp.float32)
    h1 = jnp.dot(g1_ref[...], w, preferred_element_type=jnp.float32)
    h2 = jnp.dot(g2_ref[...], w, preferred_element_type=jnp.float32)
    h = jnp.concatenate([h0, h1, h2], axis=1)
    mu = jnp.mean(h, axis=1, keepdims=True)
    var = jnp.mean(h * h, axis=1, keepdims=True) - mu * mu
    inv = lax.rsqrt(var + 1e-5)
    out_ref[...] = (h - mu) * inv * scale_ref[...] + bias_ref[...]


def _ln_tc(x, g1, g2, w, ln_scale, ln_bias):
    n, d = x.shape
    out_dim = ln_scale.shape[0]
    bt = 1000
    grid = n // bt
    return pl.pallas_call(
        _ln_body,
        grid=(grid,),
        in_specs=[
            pl.BlockSpec((bt, d), lambda i: (i, 0)),
            pl.BlockSpec((bt, d), lambda i: (i, 0)),
            pl.BlockSpec((bt, d), lambda i: (i, 0)),
            pl.BlockSpec((d, d), lambda i: (0, 0)),
            pl.BlockSpec((1, out_dim), lambda i: (0, 0)),
            pl.BlockSpec((1, out_dim), lambda i: (0, 0)),
        ],
        out_specs=pl.BlockSpec((bt, out_dim), lambda i: (i, 0)),
        out_shape=jax.ShapeDtypeStruct((n, out_dim), jnp.float32),
    )(x, g1, g2, w, ln_scale.reshape(1, -1), ln_bias.reshape(1, -1))


def kernel(x, adj1_indices, adj1_values, adj2_indices, adj2_values,
           W, ln_scale, ln_bias):
    e = adj1_values.shape[0]
    groups = -(-e // (_NS * _G * _K))
    chunks_total = groups * _NS * _G
    epad = chunks_total * _K
    pad = epad - e

    def prep(indices, values):
        dst = indices[0]
        src = indices[1]
        if pad:
            zi = jnp.zeros((pad,), jnp.int32)
            src = jnp.concatenate([src, zi])
            dst = jnp.concatenate([dst, zi])
            values = jnp.concatenate([values, jnp.zeros((pad,), jnp.float32)])
        idx = jnp.stack([src.reshape(chunks_total, _K),
                         dst.reshape(chunks_total, _K)], axis=1)
        return idx, values.reshape(chunks_total, _K)

    i1, v1 = prep(adj1_indices, adj1_values)
    i2, v2 = prep(adj2_indices, adj2_values)
    g1, g2 = _sc_spmm(x, i1, v1, i2, v2)
    return _ln_tc(x, g1, g2, W, ln_scale, ln_bias)


# speedup vs baseline: 3.7328x; 1.0310x over previous
import functools

import jax
import jax.numpy as jnp
from jax import lax
from jax.experimental import pallas as pl
from jax.experimental.pallas import tpu as pltpu
from jax.experimental.pallas import tpu_sc as plsc

_NC = 2
_NS = 16
_L = 16
_K = 128

_SPLAT_DNUMS = lax.GatherDimensionNumbers(
    offset_dims=(), collapsed_slice_dims=(0,), start_index_map=(0,))


_G = 16
_NB = 2
_SUB = _K // _L


def _sc_spmm(x, idx1, val1, idx2, val2):
    n, d = x.shape
    chunks_total = idx1.shape[0]
    pt = (chunks_total // _NS) * _K
    groups = chunks_total // (_NS * _G)
    npt = -(-n // (_NS * _K)) * _K
    n_pad = npt * _NS
    assert chunks_total == groups * _NS * _G and d % _L == 0

    mesh = plsc.VectorSubcoreMesh(core_axis_name="c", subcore_axis_name="s")

    @functools.partial(
        pl.kernel,
        mesh=mesh,
        out_type=(
            jax.ShapeDtypeStruct((n_pad, d), jnp.float32),
            jax.ShapeDtypeStruct((n_pad, d), jnp.float32),
        ),
        scratch_types=[
            pltpu.VMEM((_G, 2, _K), jnp.int32),
            pltpu.VMEM((_G, _K), jnp.float32),
            *[pltpu.VMEM((_K, d), jnp.float32) for _ in range(_NB)],
            *[pltpu.SemaphoreType.DMA for _ in range(_NB)],
            pltpu.SemaphoreType.DMA,
            pltpu.VMEM_SHARED((n_pad, d), jnp.float32),
        ],
    )
    def spmm_kernel(x_hbm, i1, v1, i2, v2, g1, g2,
                    idx_v, val_v, *rest):
        rows = rest[:_NB]
        gsem = rest[_NB:2 * _NB]
        ssem = rest[2 * _NB]
        acc = rest[2 * _NB + 1]
        r0 = rows[0]
        cid = lax.axis_index("c")
        tid = lax.axis_index("s")

        zero = jnp.zeros((_L,), jnp.float32)

        @pl.loop(0, _K)
        def _(r):
            for j in range(d // _L):
                r0[r, pl.ds(j * _L, _L)] = zero

        def gather_start(b, j):
            for o in range(_SUB):
                iv = idx_v[j, 0, pl.ds(o * _L, _L)]
                pltpu.async_copy(x_hbm.at[iv],
                                 rows[b].at[pl.ds(o * _L, _L)], gsem[b])

        def gather_wait(b):
            iv0 = idx_v[0, 0, pl.ds(0, _L)]
            for o in range(_SUB):
                pltpu.make_async_copy(
                    x_hbm.at[iv0],
                    rows[b].at[pl.ds(o * _L, _L)], gsem[b]).wait()

        def scatter_sync(b, j):
            pltpu.async_copy(rows[b], acc.at[idx_v.at[j, 1]], ssem,
                             add=True).wait()

        def scale(b, j):
            rb = rows[b]

            @pl.loop(0, _K // _L)
            def _(gg):
                gbase = gg * _L
                val16 = val_v[j, pl.ds(gbase, _L)]
                for l in range(_L):
                    vs = lax.gather(
                        val16, jnp.full((_L, 1), l, jnp.int32),
                        _SPLAT_DNUMS, (1,),
                        mode=lax.GatherScatterMode.PROMISE_IN_BOUNDS)
                    for jj in range(d // _L):
                        sl = pl.ds(jj * _L, _L)
                        rb[gbase + l, sl] = rb[gbase + l, sl] * vs

        def process(i_hbm, v_hbm, out_hbm):
            row0 = tid * npt
            for i in range(npt // _K):
                pltpu.sync_copy(r0, acc.at[pl.ds(row0 + i * _K, _K)])
            plsc.subcore_barrier()

            gchunk0 = tid * (groups * _G)

            @pl.loop(0, groups)
            def _(g):
                row = pl.ds(gchunk0 + g * _G, _G)
                pltpu.sync_copy(i_hbm.at[row], idx_v)
                pltpu.sync_copy(v_hbm.at[row], val_v)
                gather_start(0, 0)
                gather_start(1, 1)

                @pl.loop(0, _G, step=2)
                def _(j):
                    gather_wait(0)
                    scale(0, j)
                    scatter_sync(0, j)

                    @pl.when(j + 2 < _G)
                    def _():
                        gather_start(0, j + 2)

                    gather_wait(1)
                    scale(1, j + 1)
                    scatter_sync(1, j + 1)

                    @pl.when(j + 3 < _G)
                    def _():
                        gather_start(1, j + 3)

            plsc.subcore_barrier()
            for i in range(npt // _K):
                pltpu.sync_copy(acc.at[pl.ds(row0 + i * _K, _K)], r0)
                pltpu.sync_copy(r0, out_hbm.at[pl.ds(row0 + i * _K, _K)])

        @pl.when(cid == 0)
        def _():
            process(i1, v1, g1)

        @pl.when(cid == 1)
        def _():
            process(i2, v2, g2)

    return spmm_kernel(x, idx1, val1, idx2, val2)


def _ln_body(x_ref, g1_ref, g2_ref, w_ref, scale_ref, bias_ref, out_ref):
    w = w_ref[...]
    h0 = jnp.dot(x_ref[...], w, preferred_element_type=jnp.float32)
    h1 = jnp.dot(g1_ref[...], w, preferred_element_type=jnp.float32)
    h2 = jnp.dot(g2_ref[...], w, preferred_element_type=jnp.float32)
    h = jnp.concatenate([h0, h1, h2], axis=1)
    mu = jnp.mean(h, axis=1, keepdims=True)
    var = jnp.mean(h * h, axis=1, keepdims=True) - mu * mu
    inv = lax.rsqrt(var + 1e-5)
    out_ref[...] = (h - mu) * inv * scale_ref[...] + bias_ref[...]


def _ln_tc(x, g1, g2, w, ln_scale, ln_bias):
    n, d = x.shape
    out_dim = ln_scale.shape[0]
    bt = 1000
    grid = n // bt
    return pl.pallas_call(
        _ln_body,
        grid=(grid,),
        in_specs=[
            pl.BlockSpec((bt, d), lambda i: (i, 0)),
            pl.BlockSpec((bt, d), lambda i: (i, 0)),
            pl.BlockSpec((bt, d), lambda i: (i, 0)),
            pl.BlockSpec((d, d), lambda i: (0, 0)),
            pl.BlockSpec((1, out_dim), lambda i: (0, 0)),
            pl.BlockSpec((1, out_dim), lambda i: (0, 0)),
        ],
        out_specs=pl.BlockSpec((bt, out_dim), lambda i: (i, 0)),
        out_shape=jax.ShapeDtypeStruct((n, out_dim), jnp.float32),
    )(x, g1, g2, w, ln_scale.reshape(1, -1), ln_bias.reshape(1, -1))


def kernel(x, adj1_indices, adj1_values, adj2_indices, adj2_values,
           W, ln_scale, ln_bias):
    e = adj1_values.shape[0]
    groups = -(-e // (_NS * _G * _K))
    chunks_total = groups * _NS * _G
    epad = chunks_total * _K
    pad = epad - e

    def prep(indices, values):
        dst = indices[0]
        src = indices[1]
        if pad:
            zi = jnp.zeros((pad,), jnp.int32)
            src = jnp.concatenate([src, zi])
            dst = jnp.concatenate([dst, zi])
            values = jnp.concatenate([values, jnp.zeros((pad,), jnp.float32)])
        idx = jnp.stack([src.reshape(chunks_total, _K),
                         dst.reshape(chunks_total, _K)], axis=1)
        return idx, values.reshape(chunks_total, _K)

    i1, v1 = prep(adj1_indices, adj1_values)
    i2, v2 = prep(adj2_indices, adj2_values)
    g1, g2 = _sc_spmm(x, i1, v1, i2, v2)
    return _ln_tc(x, g1, g2, W, ln_scale, ln_bias)
